# baseline (speedup 1.0000x reference)
import functools

import jax
import jax.numpy as jnp
from jax import lax
from jax.experimental import pallas as pl
from jax.experimental.pallas import tpu as pltpu
from jax.experimental.pallas import tpu_sc as plsc

B = 16
L = 4096
D = 1024
NC = 2
NS = 16
DH = D // NC
CHUNK = 32
NLANE = 16
NCG = DH // NLANE


def _body(features, eff_hbm, inv_hbm, out, eff_v, inv_v, buf, part, red, shared):
    c = lax.axis_index("c")
    s = lax.axis_index("s")
    col0 = c * DH

    pltpu.sync_copy(eff_hbm, eff_v)
    pltpu.sync_copy(inv_hbm, inv_v)

    iota = lax.iota(jnp.int32, NLANE)
    eff_vec = eff_v[...]

    zero = jnp.zeros((NLANE,), jnp.float32)

    def zero_row(i, _):
        def zero_cg(g, _):
            part[i, pl.ds(g * NLANE, NLANE)] = zero
            return 0
        return lax.fori_loop(0, NCG, zero_cg, 0)

    lax.fori_loop(0, B, zero_row, 0)

    def accum_chunk(i, nrows, t0, lo):
        def per_cg(g, _):
            base = g * NLANE
            acc = part[i, pl.ds(base, NLANE)]

            def per_row(j, acc):
                scale = jnp.where(t0 + j >= lo, 1.0, 0.0)
                return acc + buf[j, pl.ds(base, NLANE)] * scale

            acc = lax.fori_loop(0, nrows, per_row, acc)
            part[i, pl.ds(base, NLANE)] = acc
            return 0

        lax.fori_loop(0, NCG, per_cg, 0)

    for i in range(B):
        eff_i = jnp.sum(jnp.where(iota == i, eff_vec, 0))
        r0 = (s * eff_i) // NS
        r1 = ((s + 1) * eff_i) // NS
        cnt = r1 - r0
        fc = cnt // CHUNK
        rem = cnt - fc * CHUNK

        def full_chunk(k, _):
            pltpu.sync_copy(
                features.at[i, pl.ds(r0 + k * CHUNK, CHUNK), pl.ds(col0, DH)],
                buf,
            )
            accum_chunk(i, CHUNK, r0, r0)
            return 0

        lax.fori_loop(0, fc, full_chunk, 0)

        @pl.when(rem > 0)
        def _():
            t0 = jnp.maximum(r1 - CHUNK, 0)
            pltpu.sync_copy(
                features.at[i, pl.ds(t0, CHUNK), pl.ds(col0, DH)], buf
            )
            nrows = jnp.minimum(r1 - t0, CHUNK)
            accum_chunk(i, nrows, t0, r0 + fc * CHUNK)

    pltpu.sync_copy(part, shared.at[s])
    plsc.subcore_barrier()

    inv_vec = inv_v[...]
    inv_s = jnp.sum(jnp.where(iota == s, inv_vec, 0.0))

    def red_init(g, _):
        red[pl.ds(g * NLANE, NLANE)] = zero
        return 0

    lax.fori_loop(0, NCG, red_init, 0)

    for w in range(NS):
        pltpu.sync_copy(shared.at[w, s], buf.at[0])

        def red_add(g, _):
            base = g * NLANE
            red[pl.ds(base, NLANE)] = red[pl.ds(base, NLANE)] + buf[0, pl.ds(base, NLANE)]
            return 0

        lax.fori_loop(0, NCG, red_add, 0)

    def red_scale(g, _):
        base = g * NLANE
        red[pl.ds(base, NLANE)] = red[pl.ds(base, NLANE)] * inv_s
        return 0

    lax.fori_loop(0, NCG, red_scale, 0)

    pltpu.sync_copy(red, out.at[s, pl.ds(col0, DH)])


@jax.jit
def kernel(features, lengths):
    eff = jnp.minimum(jnp.where(lengths <= 0, L, lengths), L).astype(jnp.int32)
    inv = (1.0 / eff.astype(jnp.float32))

    mesh = plsc.VectorSubcoreMesh(core_axis_name="c", subcore_axis_name="s")
    run = pl.kernel(
        _body,
        out_type=jax.ShapeDtypeStruct((B, D), jnp.float32),
        mesh=mesh,
        scratch_types=[
            pltpu.VMEM((B,), jnp.int32),
            pltpu.VMEM((B,), jnp.float32),
            pltpu.VMEM((CHUNK, DH), jnp.float32),
            pltpu.VMEM((B, DH), jnp.float32),
            pltpu.VMEM((DH,), jnp.float32),
            pltpu.VMEM_SHARED((NS, B, DH), jnp.float32),
        ],
        compiler_params=pltpu.CompilerParams(
            use_tc_tiling_on_sc=False, needs_layout_passes=False
        ),
    )
    return run(features, eff, inv)


# speedup vs baseline: 1.2997x; 1.2997x over previous
import jax
import jax.numpy as jnp
from jax import lax
from jax.experimental import pallas as pl
from jax.experimental.pallas import tpu as pltpu
from jax.experimental.pallas import tpu_sc as plsc

B = 16
L = 4096
D = 1024
NC = 2
NS = 16
DH = D // NC
CHUNK = 32
NLANE = 16
NCG = DH // NLANE


def _body(features, eff_hbm, inv_hbm, out, eff_v, inv_v, buf, part, red, shared, sems):
    c = lax.axis_index("c")
    s = lax.axis_index("s")
    col0 = c * DH

    pltpu.sync_copy(eff_hbm, eff_v)
    pltpu.sync_copy(inv_hbm, inv_v)

    iota = lax.iota(jnp.int32, NLANE)
    eff_vec = eff_v[...]
    zero = jnp.zeros((NLANE,), jnp.float32)

    def zero_row(i, _):
        def zero_cg(g, _):
            part[i, pl.ds(g * NLANE, NLANE)] = zero
            return 0
        return lax.fori_loop(0, NCG, zero_cg, 0)

    lax.fori_loop(0, B, zero_row, 0)

    def per_example(i, _):
        eff_i = jnp.sum(jnp.where(iota == i, eff_vec, 0))
        r0 = (s * eff_i) // NS
        r1 = ((s + 1) * eff_i) // NS
        cnt = r1 - r0
        nchunks = (cnt + CHUNK - 1) // CHUNK
        tmax = jnp.maximum(r1 - CHUNK, 0)
        last = nchunks - 1

        def start_of(k):
            return jnp.where(k == last, tmax, r0 + k * CHUNK)

        def issue(k):
            par = k & 1
            return pltpu.async_copy(
                features.at[i, pl.ds(start_of(k), CHUNK), pl.ds(col0, DH)],
                buf.at[par],
                sems.at[par],
            )

        @pl.when(nchunks > 0)
        def _():
            issue(0)

        def chunk_step(k, _):
            par = k & 1

            @pl.when(k + 1 < nchunks)
            def _():
                issue(k + 1)

            pltpu.make_async_copy(
                features.at[i, pl.ds(start_of(k), CHUNK), pl.ds(col0, DH)],
                buf.at[par],
                sems.at[par],
            ).wait()

            @pl.when(k != last)
            def _():
                def per_cg(g, _):
                    base = g * NLANE
                    acc = part[i, pl.ds(base, NLANE)]
                    for j in range(CHUNK):
                        acc = acc + buf[par, j, pl.ds(base, NLANE)]
                    part[i, pl.ds(base, NLANE)] = acc
                    return 0

                lax.fori_loop(0, NCG, per_cg, 0)

            @pl.when(k == last)
            def _():
                lo = r0 + last * CHUNK
                scales = [
                    jnp.where((tmax + j >= lo) & (tmax + j < r1), 1.0, 0.0)
                    for j in range(CHUNK)
                ]

                def per_cg(g, _):
                    base = g * NLANE
                    acc = part[i, pl.ds(base, NLANE)]
                    for j in range(CHUNK):
                        acc = acc + buf[par, j, pl.ds(base, NLANE)] * scales[j]
                    part[i, pl.ds(base, NLANE)] = acc
                    return 0

                lax.fori_loop(0, NCG, per_cg, 0)

            return 0

        lax.fori_loop(0, nchunks, chunk_step, 0)
        return 0

    lax.fori_loop(0, B, per_example, 0)

    pltpu.sync_copy(part, shared.at[s])
    plsc.subcore_barrier()

    inv_vec = inv_v[...]
    inv_s = jnp.sum(jnp.where(iota == s, inv_vec, 0.0))

    def red_init(g, _):
        red[pl.ds(g * NLANE, NLANE)] = zero
        return 0

    lax.fori_loop(0, NCG, red_init, 0)

    def red_worker(w, _):
        pltpu.sync_copy(shared.at[w, s], buf.at[0, 0])

        def red_add(g, _):
            base = g * NLANE
            red[pl.ds(base, NLANE)] = red[pl.ds(base, NLANE)] + buf[0, 0, pl.ds(base, NLANE)]
            return 0

        lax.fori_loop(0, NCG, red_add, 0)
        return 0

    lax.fori_loop(0, NS, red_worker, 0)

    def red_scale(g, _):
        base = g * NLANE
        red[pl.ds(base, NLANE)] = red[pl.ds(base, NLANE)] * inv_s
        return 0

    lax.fori_loop(0, NCG, red_scale, 0)

    pltpu.sync_copy(red, out.at[s, pl.ds(col0, DH)])


@jax.jit
def kernel(features, lengths):
    eff = jnp.minimum(jnp.where(lengths <= 0, L, lengths), L).astype(jnp.int32)
    inv = (1.0 / eff.astype(jnp.float32))

    mesh = plsc.VectorSubcoreMesh(core_axis_name="c", subcore_axis_name="s")
    run = pl.kernel(
        _body,
        out_type=jax.ShapeDtypeStruct((B, D), jnp.float32),
        mesh=mesh,
        scratch_types=[
            pltpu.VMEM((B,), jnp.int32),
            pltpu.VMEM((B,), jnp.float32),
            pltpu.VMEM((2, CHUNK, DH), jnp.float32),
            pltpu.VMEM((B, DH), jnp.float32),
            pltpu.VMEM((DH,), jnp.float32),
            pltpu.VMEM_SHARED((NS, B, DH), jnp.float32),
            pltpu.SemaphoreType.DMA((2,)),
        ],
        compiler_params=pltpu.CompilerParams(
            use_tc_tiling_on_sc=False, needs_layout_passes=False
        ),
    )
    return run(features, eff, inv)


# speedup vs baseline: 3.1595x; 2.4310x over previous
import jax
import jax.numpy as jnp
from jax import lax
from jax.experimental import pallas as pl
from jax.experimental.pallas import tpu as pltpu
from jax.experimental.pallas import tpu_sc as plsc

B = 16
L = 4096
D = 1024
NC = 2
NS = 16
DH = D // NC
CHUNK = 32
NLANE = 16
NCG = DH // NLANE


def _body(features, eff_hbm, inv_hbm, out, eff_v, inv_v, buf, part, red, shared,
          final, sems):
    c = lax.axis_index("c")
    s = lax.axis_index("s")
    col0 = c * DH

    pltpu.sync_copy(eff_hbm, eff_v)
    pltpu.sync_copy(inv_hbm, inv_v)

    iota = lax.iota(jnp.int32, NLANE)
    eff_vec = eff_v[...]
    zero = jnp.zeros((NLANE,), jnp.float32)

    def zero_row(i, _):
        def zero_cg(g, _):
            part[i, pl.ds(g * NLANE, NLANE)] = zero
            return 0
        return lax.fori_loop(0, NCG, zero_cg, 0)

    lax.fori_loop(0, B, zero_row, 0)

    def per_example(i, _):
        eff_i = jnp.sum(jnp.where(iota == i, eff_vec, 0))
        nblk = (eff_i + 7) // 8
        r0 = 8 * ((s * nblk) // NS)
        rb1 = 8 * (((s + 1) * nblk) // NS)
        cnt = rb1 - r0
        nchunks = (cnt + CHUNK - 1) // CHUNK
        tmax = jnp.maximum(rb1 - CHUNK, 0)
        last = nchunks - 1
        hi = jnp.minimum(rb1, eff_i)

        def start_of(k):
            return pl.multiple_of(jnp.where(k == last, tmax, r0 + k * CHUNK), 8)

        def issue(k):
            par = k & 1
            return pltpu.async_copy(
                features.at[i, pl.ds(start_of(k), CHUNK), pl.ds(col0, DH)],
                buf.at[par],
                sems.at[par],
            )

        @pl.when(nchunks > 0)
        def _():
            issue(0)

        def chunk_step(k, _):
            par = k & 1

            @pl.when(k + 1 < nchunks)
            def _():
                issue(k + 1)

            pltpu.make_async_copy(
                features.at[i, pl.ds(start_of(k), CHUNK), pl.ds(col0, DH)],
                buf.at[par],
                sems.at[par],
            ).wait()

            @pl.when(k != last)
            def _():
                def per_cg(g, _):
                    base = g * NLANE
                    acc = part[i, pl.ds(base, NLANE)]
                    for j in range(CHUNK):
                        acc = acc + buf[par, j, pl.ds(base, NLANE)]
                    part[i, pl.ds(base, NLANE)] = acc
                    return 0

                lax.fori_loop(0, NCG, per_cg, 0)

            @pl.when(k == last)
            def _():
                lo = r0 + last * CHUNK
                scales = [
                    jnp.where((tmax + j >= lo) & (tmax + j < hi), 1.0, 0.0)
                    for j in range(CHUNK)
                ]

                def per_cg(g, _):
                    base = g * NLANE
                    acc = part[i, pl.ds(base, NLANE)]
                    for j in range(CHUNK):
                        acc = acc + buf[par, j, pl.ds(base, NLANE)] * scales[j]
                    part[i, pl.ds(base, NLANE)] = acc
                    return 0

                lax.fori_loop(0, NCG, per_cg, 0)

            return 0

        lax.fori_loop(0, nchunks, chunk_step, 0)
        return 0

    lax.fori_loop(0, B, per_example, 0)

    pltpu.sync_copy(part, shared.at[s])
    plsc.subcore_barrier()

    inv_vec = inv_v[...]
    inv_s = jnp.sum(jnp.where(iota == s, inv_vec, 0.0))

    def red_init(g, _):
        red[pl.ds(g * NLANE, NLANE)] = zero
        return 0

    lax.fori_loop(0, NCG, red_init, 0)

    def red_worker(w, _):
        pltpu.sync_copy(shared.at[w, s], buf.at[0, 0])

        def red_add(g, _):
            base = g * NLANE
            red[pl.ds(base, NLANE)] = red[pl.ds(base, NLANE)] + buf[0, 0, pl.ds(base, NLANE)]
            return 0

        lax.fori_loop(0, NCG, red_add, 0)
        return 0

    lax.fori_loop(0, NS, red_worker, 0)

    def red_scale(g, _):
        base = g * NLANE
        red[pl.ds(base, NLANE)] = red[pl.ds(base, NLANE)] * inv_s
        return 0

    lax.fori_loop(0, NCG, red_scale, 0)

    pltpu.sync_copy(red, final.at[s])
    plsc.subcore_barrier()

    @pl.when(s == 0)
    def _():
        pltpu.sync_copy(final, out.at[:, pl.ds(col0, DH)])


@jax.jit
def kernel(features, lengths):
    eff = jnp.minimum(jnp.where(lengths <= 0, L, lengths), L).astype(jnp.int32)
    inv = (1.0 / eff.astype(jnp.float32))

    mesh = plsc.VectorSubcoreMesh(core_axis_name="c", subcore_axis_name="s")
    run = pl.kernel(
        _body,
        out_type=jax.ShapeDtypeStruct((B, D), jnp.float32),
        mesh=mesh,
        scratch_types=[
            pltpu.VMEM((B,), jnp.int32),
            pltpu.VMEM((B,), jnp.float32),
            pltpu.VMEM((2, CHUNK, DH), jnp.float32),
            pltpu.VMEM((B, DH), jnp.float32),
            pltpu.VMEM((DH,), jnp.float32),
            pltpu.VMEM_SHARED((NS, B, DH), jnp.float32),
            pltpu.VMEM_SHARED((B, DH), jnp.float32),
            pltpu.SemaphoreType.DMA((2,)),
        ],
        compiler_params=pltpu.CompilerParams(needs_layout_passes=False),
    )
    return run(features, eff, inv)


# speedup vs baseline: 3.8012x; 1.2031x over previous
import jax
import jax.numpy as jnp
from jax import lax
from jax.experimental import pallas as pl
from jax.experimental.pallas import tpu as pltpu
from jax.experimental.pallas import tpu_sc as plsc

B = 16
L = 4096
D = 1024
NC = 2
NS = 16
DH = D // NC
CHUNK = 32
NLANE = 16
NCG = DH // NLANE


def _body(features, eff_hbm, inv_hbm, out, eff_v, inv_v, buf, part, red, shared,
          final, sems):
    c = lax.axis_index("c")
    s = lax.axis_index("s")
    col0 = c * DH

    pltpu.sync_copy(eff_hbm, eff_v)
    pltpu.sync_copy(inv_hbm, inv_v)

    iota = lax.iota(jnp.int32, NLANE)
    eff_vec = eff_v[...]
    zero = jnp.zeros((NLANE,), jnp.float32)

    def zero_row(i, _):
        def zero_cg(g, _):
            part[i, pl.ds(g * NLANE, NLANE)] = zero
            return 0
        return lax.fori_loop(0, NCG, zero_cg, 0)

    lax.fori_loop(0, B, zero_row, 0)

    nblk_v = (eff_vec + 7) // 8
    r0_v = 8 * ((s * nblk_v) // NS)
    rb1_v = 8 * (((s + 1) * nblk_v) // NS)
    nch_v = (rb1_v - r0_v + CHUNK - 1) // CHUNK
    hi_v = jnp.minimum(rb1_v, eff_vec)
    total = jnp.sum(nch_v)

    def geti(v, i):
        return jnp.sum(jnp.where(iota == i, v, 0))

    def advance(i, k):
        def cond(st):
            i_, k_ = st
            return (i_ < B) & (k_ >= geti(nch_v, i_))

        def step(st):
            i_, _ = st
            return (i_ + 1, 0)

        return lax.while_loop(cond, step, (i, k))

    def chunk_start(i, k):
        r0 = geti(r0_v, i)
        rb1 = geti(rb1_v, i)
        islast = k == geti(nch_v, i) - 1
        start = jnp.where(
            islast, jnp.maximum(rb1 - CHUNK, 0), r0 + k * CHUNK
        )
        return pl.multiple_of(start, 8)

    def issue(i, k, par):
        return pltpu.async_copy(
            features.at[i, pl.ds(chunk_start(i, k), CHUNK), pl.ds(col0, DH)],
            buf.at[par],
            sems.at[par],
        )

    first = advance(0, 0)

    @pl.when(total > 0)
    def _():
        issue(first[0], first[1], 0)

    def chunk_step(m, st):
        i, k = st
        par = m & 1
        nxt = advance(i, k + 1)

        @pl.when(m + 1 < total)
        def _():
            issue(nxt[0], nxt[1], (m + 1) & 1)

        pltpu.make_async_copy(
            features.at[i, pl.ds(chunk_start(i, k), CHUNK), pl.ds(col0, DH)],
            buf.at[par],
            sems.at[par],
        ).wait()

        start = chunk_start(i, k)
        lo = geti(r0_v, i) + k * CHUNK
        hi = jnp.where(k == geti(nch_v, i) - 1, geti(hi_v, i), lo + CHUNK)
        scales = [
            jnp.where((start + j >= lo) & (start + j < hi), 1.0, 0.0)
            for j in range(CHUNK)
        ]

        def per_cg(g, _):
            base = g * NLANE
            acc = part[i, pl.ds(base, NLANE)]
            for j in range(CHUNK):
                acc = acc + buf[par, j, pl.ds(base, NLANE)] * scales[j]
            part[i, pl.ds(base, NLANE)] = acc
            return 0

        lax.fori_loop(0, NCG, per_cg, 0)
        return nxt

    lax.fori_loop(0, total, chunk_step, first)

    pltpu.sync_copy(part, shared.at[s])
    plsc.subcore_barrier()

    inv_vec = inv_v[...]
    inv_s = jnp.sum(jnp.where(iota == s, inv_vec, 0.0))

    def red_init(g, _):
        red[pl.ds(g * NLANE, NLANE)] = zero
        return 0

    lax.fori_loop(0, NCG, red_init, 0)

    def red_worker(w, _):
        pltpu.sync_copy(shared.at[w, s], buf.at[0, 0])

        def red_add(g, _):
            base = g * NLANE
            red[pl.ds(base, NLANE)] = red[pl.ds(base, NLANE)] + buf[0, 0, pl.ds(base, NLANE)]
            return 0

        lax.fori_loop(0, NCG, red_add, 0)
        return 0

    lax.fori_loop(0, NS, red_worker, 0)

    def red_scale(g, _):
        base = g * NLANE
        red[pl.ds(base, NLANE)] = red[pl.ds(base, NLANE)] * inv_s
        return 0

    lax.fori_loop(0, NCG, red_scale, 0)

    pltpu.sync_copy(red, final.at[s])
    plsc.subcore_barrier()

    @pl.when(s == 0)
    def _():
        pltpu.sync_copy(final, out.at[:, pl.ds(col0, DH)])


@jax.jit
def kernel(features, lengths):
    eff = jnp.minimum(jnp.where(lengths <= 0, L, lengths), L).astype(jnp.int32)
    inv = (1.0 / eff.astype(jnp.float32))

    mesh = plsc.VectorSubcoreMesh(core_axis_name="c", subcore_axis_name="s")
    run = pl.kernel(
        _body,
        out_type=jax.ShapeDtypeStruct((B, D), jnp.float32),
        mesh=mesh,
        scratch_types=[
            pltpu.VMEM((B,), jnp.int32),
            pltpu.VMEM((B,), jnp.float32),
            pltpu.VMEM((2, CHUNK, DH), jnp.float32),
            pltpu.VMEM((B, DH), jnp.float32),
            pltpu.VMEM((DH,), jnp.float32),
            pltpu.VMEM_SHARED((NS, B, DH), jnp.float32),
            pltpu.VMEM_SHARED((B, DH), jnp.float32),
            pltpu.SemaphoreType.DMA((2,)),
        ],
        compiler_params=pltpu.CompilerParams(needs_layout_passes=False),
    )
    return run(features, eff, inv)


# speedup vs baseline: 4.1779x; 1.0991x over previous
import jax
import jax.numpy as jnp
from jax import lax
from jax.experimental import pallas as pl
from jax.experimental.pallas import tpu as pltpu
from jax.experimental.pallas import tpu_sc as plsc

B = 16
L = 4096
D = 1024
NC = 2
NS = 16
DH = D // NC
CHUNK = 32
NLANE = 16
NCG = DH // NLANE


def _body(features, eff_hbm, inv_hbm, out, eff_v, inv_v, buf, part, red, shared,
          final, sems):
    c = lax.axis_index("c")
    s = lax.axis_index("s")
    col0 = c * DH

    pltpu.sync_copy(eff_hbm, eff_v)
    pltpu.sync_copy(inv_hbm, inv_v)

    iota = lax.iota(jnp.int32, NLANE)
    eff_vec = eff_v[...]
    zero = jnp.zeros((NLANE,), jnp.float32)

    def zero_row(i, _):
        def zero_cg(g, _):
            part[i, pl.ds(g * NLANE, NLANE)] = zero
            return 0
        return lax.fori_loop(0, NCG, zero_cg, 0)

    lax.fori_loop(0, B, zero_row, 0)

    nblk_v = (eff_vec + 7) // 8
    r0_v = 8 * ((s * nblk_v) // NS)
    rb1_v = 8 * (((s + 1) * nblk_v) // NS)
    nch_v = (rb1_v - r0_v + CHUNK - 1) // CHUNK
    hi_v = jnp.minimum(rb1_v, eff_vec)
    total = jnp.sum(nch_v)

    def geti(v, i):
        return jnp.sum(jnp.where(iota == i, v, 0))

    def advance(i, k):
        def cond(st):
            i_, k_ = st
            return (i_ < B) & (k_ >= geti(nch_v, i_))

        def step(st):
            i_, _ = st
            return (i_ + 1, 0)

        return lax.while_loop(cond, step, (i, k))

    def chunk_start(i, k):
        r0 = geti(r0_v, i)
        rb1 = geti(rb1_v, i)
        islast = k == geti(nch_v, i) - 1
        start = jnp.where(
            islast, jnp.maximum(rb1 - CHUNK, 0), r0 + k * CHUNK
        )
        return pl.multiple_of(start, 8)

    def issue(i, k, par):
        return pltpu.async_copy(
            features.at[i, pl.ds(chunk_start(i, k), CHUNK), pl.ds(col0, DH)],
            buf.at[par],
            sems.at[par],
        )

    c0 = advance(0, 0)
    c1 = advance(c0[0], c0[1] + 1)
    c2 = advance(c1[0], c1[1] + 1)
    c3 = advance(c2[0], c2[1] + 1)
    for depth, cc in enumerate((c0, c1, c2)):
        @pl.when(total > depth)
        def _(cc=cc, depth=depth):
            issue(cc[0], cc[1], depth)

    def chunk_step(m, st):
        i, k, n1, n2, n3 = st
        par = m & 3

        @pl.when(m + 3 < total)
        def _():
            issue(n3[0], n3[1], (m + 3) & 3)

        pltpu.make_async_copy(
            features.at[i, pl.ds(chunk_start(i, k), CHUNK), pl.ds(col0, DH)],
            buf.at[par],
            sems.at[par],
        ).wait()

        start = chunk_start(i, k)
        lo = geti(r0_v, i) + k * CHUNK
        hi = jnp.where(k == geti(nch_v, i) - 1, geti(hi_v, i), lo + CHUNK)
        scales = [
            jnp.where((start + j >= lo) & (start + j < hi), 1.0, 0.0)
            for j in range(CHUNK)
        ]

        def per_cg(g, _):
            base = g * NLANE
            acc = part[i, pl.ds(base, NLANE)]
            for j in range(CHUNK):
                acc = acc + buf[par, j, pl.ds(base, NLANE)] * scales[j]
            part[i, pl.ds(base, NLANE)] = acc
            return 0

        lax.fori_loop(0, NCG, per_cg, 0)
        return (n1[0], n1[1], n2, n3, advance(n3[0], n3[1] + 1))

    lax.fori_loop(0, total, chunk_step, (c0[0], c0[1], c1, c2, c3))

    pltpu.sync_copy(part, shared.at[s])
    plsc.subcore_barrier()

    inv_vec = inv_v[...]
    inv_s = jnp.sum(jnp.where(iota == s, inv_vec, 0.0))

    pltpu.sync_copy(shared.at[:, s], part)

    def red_cg(g, _):
        base = g * NLANE
        acc = part[0, pl.ds(base, NLANE)]
        for w in range(1, NS):
            acc = acc + part[w, pl.ds(base, NLANE)]
        red[pl.ds(base, NLANE)] = acc * inv_s
        return 0

    lax.fori_loop(0, NCG, red_cg, 0)

    pltpu.sync_copy(red, final.at[s])
    plsc.subcore_barrier()

    @pl.when(s == 0)
    def _():
        pltpu.sync_copy(final, out.at[:, pl.ds(col0, DH)])


@jax.jit
def kernel(features, lengths):
    eff = jnp.minimum(jnp.where(lengths <= 0, L, lengths), L).astype(jnp.int32)
    inv = (1.0 / eff.astype(jnp.float32))

    mesh = plsc.VectorSubcoreMesh(core_axis_name="c", subcore_axis_name="s")
    run = pl.kernel(
        _body,
        out_type=jax.ShapeDtypeStruct((B, D), jnp.float32),
        mesh=mesh,
        scratch_types=[
            pltpu.VMEM((B,), jnp.int32),
            pltpu.VMEM((B,), jnp.float32),
            pltpu.VMEM((4, CHUNK, DH), jnp.float32),
            pltpu.VMEM((B, DH), jnp.float32),
            pltpu.VMEM((DH,), jnp.float32),
            pltpu.VMEM_SHARED((NS, B, DH), jnp.float32),
            pltpu.VMEM_SHARED((B, DH), jnp.float32),
            pltpu.SemaphoreType.DMA((4,)),
        ],
        compiler_params=pltpu.CompilerParams(needs_layout_passes=False),
    )
    return run(features, eff, inv)
